# baseline (speedup 1.0000x reference)
import functools

import jax
import jax.numpy as jnp
from jax import lax
from jax.experimental import pallas as pl
from jax.experimental.pallas import tpu as pltpu
from jax.experimental.pallas import tpu_sc as plsc

_N = 10000
_E = 320000
_FIN = 128
_H = 128
_OUT = 8

_NC = 2
_NS = 16
_NW = _NC * _NS
_EPW = _E // _NW
_CHUNK = 125
_NCHUNK = _EPW // _CHUNK
_GROUP = 10
_NGRP = _NCHUNK // _GROUP
_RPT = 624
_RPT_LAST = _N - 15 * _RPT

@functools.cache
def _make_segsum_sc():
    mesh = plsc.VectorSubcoreMesh(
        core_axis_name="c", subcore_axis_name="s",
        num_cores=_NC, num_subcores=_NS)

    @functools.partial(
        pl.kernel,
        out_type=jax.ShapeDtypeStruct((_NC, _N, _H), jnp.float32),
        mesh=mesh,
        scratch_types=[
            pltpu.VMEM((2, _GROUP, _CHUNK), jnp.int32),
            pltpu.VMEM((2, _GROUP, _CHUNK), jnp.int32),
            pltpu.VMEM((2, _CHUNK, _H), jnp.float32),
            pltpu.VMEM_SHARED((_N, _H), jnp.float32),
            pltpu.SemaphoreType.DMA,
            pltpu.SemaphoreType.DMA,
        ],
    )
    def segsum_sc(h_hbm, src_hbm, dst_hbm, zeros_hbm, out_hbm,
                  src_v, dst_v, rows_v, acc, rsem, isem):
        c = lax.axis_index("c")
        s = lax.axis_index("s")
        wid = s * _NC + c
        r0 = pl.multiple_of(s * _RPT, 8)

        pltpu.sync_copy(src_hbm.at[wid].at[0], src_v.at[0])
        pltpu.sync_copy(dst_hbm.at[wid].at[0], dst_v.at[0])
        pltpu.async_copy(h_hbm.at[src_v.at[0].at[0]], rows_v.at[0], rsem)
        pltpu.async_copy(src_hbm.at[wid].at[1], src_v.at[1], isem)
        pltpu.async_copy(dst_hbm.at[wid].at[1], dst_v.at[1], isem)

        @pl.when(s < _NS - 1)
        def _():
            pltpu.sync_copy(zeros_hbm.at[pl.ds(0, _RPT)],
                            acc.at[pl.ds(r0, _RPT)])

        @pl.when(s == _NS - 1)
        def _():
            pltpu.sync_copy(zeros_hbm,
                            acc.at[pl.ds(15 * _RPT, _RPT_LAST)])
        plsc.subcore_barrier()

        @pl.loop(0, _NGRP)
        def _(g):
            gp = g % 2
            gq = (g + 1) % 2
            for k in range(_GROUP):
                pltpu.make_async_copy(h_hbm.at[src_v.at[gp].at[k]],
                                      rows_v.at[k % 2], rsem).wait()
                if k + 1 < _GROUP:
                    pltpu.async_copy(h_hbm.at[src_v.at[gp].at[k + 1]],
                                     rows_v.at[(k + 1) % 2], rsem)
                else:
                    @pl.when(g + 1 < _NGRP)
                    def _():
                        pltpu.make_async_copy(src_hbm.at[wid].at[g + 1],
                                              src_v.at[gq], isem).wait()
                        pltpu.make_async_copy(dst_hbm.at[wid].at[g + 1],
                                              dst_v.at[gq], isem).wait()
                        pltpu.async_copy(h_hbm.at[src_v.at[gq].at[0]],
                                         rows_v.at[0], rsem)
                pltpu.sync_copy(rows_v.at[k % 2],
                                acc.at[dst_v.at[gp].at[k]], add=True)

            @pl.when(g + 2 < _NGRP)
            def _():
                pltpu.async_copy(src_hbm.at[wid].at[g + 2], src_v.at[gp], isem)
                pltpu.async_copy(dst_hbm.at[wid].at[g + 2], dst_v.at[gp], isem)

        plsc.subcore_barrier()

        @pl.when(s < _NS - 1)
        def _():
            pltpu.sync_copy(acc.at[pl.ds(r0, _RPT)],
                            out_hbm.at[c].at[pl.ds(r0, _RPT)])

        @pl.when(s == _NS - 1)
        def _():
            pltpu.sync_copy(acc.at[pl.ds(15 * _RPT, _RPT_LAST)],
                            out_hbm.at[c].at[pl.ds(15 * _RPT, _RPT_LAST)])

    return segsum_sc


_BN = 2000
_GRID = _N // _BN


def _leaky(z):
    return jnp.where(z >= 0, z, 0.01 * z)


def _pre_body(x_ref, w_ref, b_ref, o_ref):
    o_ref[...] = lax.dot_general(
        x_ref[...], w_ref[...], (((0,), (0,)), ((), ())),
        preferred_element_type=jnp.float32) + b_ref[...]


_pre = pl.pallas_call(
    _pre_body,
    out_shape=jax.ShapeDtypeStruct((_N, _H), jnp.float32),
)


def _convbn_body(h_ref, a_ref, w1_ref, b1_ref, w2_ref, b2_ref,
                 gam_ref, bet_ref, o_ref, g_vmem, st_vmem):
    i = pl.program_id(0)

    @pl.when(i < _GRID)
    def _():
        z = h_ref[...] + a_ref[0] + a_ref[1]
        t = _leaky(jnp.dot(z, w1_ref[...], preferred_element_type=jnp.float32)
                   + b1_ref[...])
        g = (jnp.dot(t, w2_ref[...], preferred_element_type=jnp.float32)
             + b2_ref[...])
        r = pl.multiple_of(i * _BN, 8)
        g_vmem[pl.ds(r, _BN), :] = g
        st = jnp.concatenate(
            [jnp.sum(g, axis=0, keepdims=True),
             jnp.sum(g * g, axis=0, keepdims=True)], axis=0)

        @pl.when(i == 0)
        def _():
            st_vmem[...] = st

        @pl.when(i != 0)
        def _():
            st_vmem[...] += st

    @pl.when(i >= _GRID)
    def _():
        m = st_vmem[0:1, :] * (1.0 / _N)
        v = st_vmem[1:2, :] * (1.0 / _N) - m * m
        a = gam_ref[...] * lax.rsqrt(v + 1e-5)
        r = pl.multiple_of((i - _GRID) * _BN, 8)
        o_ref[...] = g_vmem[pl.ds(r, _BN), :] * a + (bet_ref[...] - m * a)


_convbn = pl.pallas_call(
    _convbn_body,
    grid=(2 * _GRID,),
    in_specs=[
        pl.BlockSpec((_BN, _H), lambda i: (jnp.where(i < _GRID, i, _GRID - 1), 0)),
        pl.BlockSpec((_NC, _BN, _H),
                     lambda i: (0, jnp.where(i < _GRID, i, _GRID - 1), 0)),
        pl.BlockSpec((_H, _H), lambda i: (0, 0)),
        pl.BlockSpec((1, _H), lambda i: (0, 0)),
        pl.BlockSpec((_H, _H), lambda i: (0, 0)),
        pl.BlockSpec((1, _H), lambda i: (0, 0)),
        pl.BlockSpec((1, _H), lambda i: (0, 0)),
        pl.BlockSpec((1, _H), lambda i: (0, 0)),
    ],
    out_specs=pl.BlockSpec((_BN, _H),
                           lambda i: (jnp.where(i < _GRID, 0, i - _GRID), 0)),
    out_shape=jax.ShapeDtypeStruct((_N, _H), jnp.float32),
    scratch_shapes=[
        pltpu.VMEM((_N, _H), jnp.float32),
        pltpu.VMEM((2, _H), jnp.float32),
    ],
)


def _final_body(h_ref, a_ref, w1_ref, b1_ref, w2_ref, b2_ref,
                wp1_ref, bp1_ref, wp2_ref, bp2_ref, o_ref):
    z = h_ref[...] + a_ref[0] + a_ref[1]
    t = _leaky(jnp.dot(z, w1_ref[...], preferred_element_type=jnp.float32)
               + b1_ref[...])
    g = jnp.dot(t, w2_ref[...], preferred_element_type=jnp.float32) + b2_ref[...]
    t2 = _leaky(jnp.dot(g, wp1_ref[...], preferred_element_type=jnp.float32)
                + bp1_ref[...])
    o_ref[...] = jnp.dot(t2, wp2_ref[...],
                         preferred_element_type=jnp.float32) + bp2_ref[...]


_final = pl.pallas_call(
    _final_body,
    grid=(_GRID,),
    in_specs=[
        pl.BlockSpec((_BN, _H), lambda i: (i, 0)),
        pl.BlockSpec((_NC, _BN, _H), lambda i: (0, i, 0)),
        pl.BlockSpec((_H, _H), lambda i: (0, 0)),
        pl.BlockSpec((1, _H), lambda i: (0, 0)),
        pl.BlockSpec((_H, _H), lambda i: (0, 0)),
        pl.BlockSpec((1, _H), lambda i: (0, 0)),
        pl.BlockSpec((_H, _H), lambda i: (0, 0)),
        pl.BlockSpec((1, _H), lambda i: (0, 0)),
        pl.BlockSpec((_H, _OUT), lambda i: (0, 0)),
        pl.BlockSpec((1, _OUT), lambda i: (0, 0)),
    ],
    out_specs=pl.BlockSpec((_BN, _OUT), lambda i: (i, 0)),
    out_shape=jax.ShapeDtypeStruct((_N, _OUT), jnp.float32),
)


def kernel(x, edge_index, W_pre, b_pre, W1_0, b1_0, W2_0, b2_0,
           W1_1, b1_1, W2_1, b2_1, W1_2, b1_2, W2_2, b2_2,
           gamma_0, beta_0, gamma_1, beta_1, Wp1, bp1, Wp2, bp2):
    src = edge_index[0].reshape(_NW, _NGRP, _GROUP, _CHUNK)
    dst = edge_index[1].reshape(_NW, _NGRP, _GROUP, _CHUNK)
    zeros = jnp.zeros((_RPT_LAST, _H), jnp.float32)
    _segsum_sc = _make_segsum_sc()

    r1 = lambda v: v.reshape(1, -1)

    h = _pre(x, W_pre, r1(b_pre))
    for W1, b1, W2, b2, gam, bet in (
        (W1_0, b1_0, W2_0, b2_0, gamma_0, beta_0),
        (W1_1, b1_1, W2_1, b2_1, gamma_1, beta_1),
    ):
        agg = _segsum_sc(h, src, dst, zeros)
        h = _convbn(h, agg, W1, r1(b1), W2, r1(b2), r1(gam), r1(bet))

    agg = _segsum_sc(h, src, dst, zeros)
    out = _final(h, agg, W1_2, r1(b1_2), W2_2, r1(b2_2),
                 Wp1, r1(bp1), Wp2, r1(bp2))
    return out.reshape(1, -1)


# speedup vs baseline: 1.2703x; 1.2703x over previous
import functools

import jax
import jax.numpy as jnp
from jax import lax
from jax.experimental import pallas as pl
from jax.experimental.pallas import tpu as pltpu
from jax.experimental.pallas import tpu_sc as plsc

_N = 10000
_E = 320000
_FIN = 128
_H = 128
_OUT = 8

_NC = 2
_NS = 16
_NW = _NC * _NS
_EPW = _E // _NW
_CHUNK = 100
_NCHUNK = _EPW // _CHUNK
_GROUP = 10
_NGRP = _NCHUNK // _GROUP
_RPT = 624
_RPT_LAST = _N - 15 * _RPT

@functools.cache
def _make_segsum_sc():
    mesh = plsc.VectorSubcoreMesh(
        core_axis_name="c", subcore_axis_name="s",
        num_cores=_NC, num_subcores=_NS)

    @functools.partial(
        pl.kernel,
        out_type=jax.ShapeDtypeStruct((_NC, _N, _H), jnp.float32),
        mesh=mesh,
        scratch_types=[
            pltpu.VMEM((2, _GROUP, _CHUNK), jnp.int32),
            pltpu.VMEM((2, _GROUP, _CHUNK), jnp.int32),
            pltpu.VMEM((3, _CHUNK, _H), jnp.float32),
            pltpu.VMEM_SHARED((_N, _H), jnp.float32),
            pltpu.SemaphoreType.DMA,
            pltpu.SemaphoreType.DMA,
        ],
    )
    def segsum_sc(h_hbm, src_hbm, dst_hbm, zeros_hbm, out_hbm,
                  src_v, dst_v, rows_v, acc, rsem, isem):
        c = lax.axis_index("c")
        s = lax.axis_index("s")
        wid = s * _NC + c
        r0 = pl.multiple_of(s * _RPT, 8)

        pltpu.sync_copy(src_hbm.at[wid].at[0], src_v.at[0])
        pltpu.sync_copy(dst_hbm.at[wid].at[0], dst_v.at[0])
        pltpu.async_copy(h_hbm.at[src_v.at[0].at[0]], rows_v.at[0], rsem)
        pltpu.async_copy(h_hbm.at[src_v.at[0].at[1]], rows_v.at[1], rsem)
        pltpu.async_copy(src_hbm.at[wid].at[1], src_v.at[1], isem)
        pltpu.async_copy(dst_hbm.at[wid].at[1], dst_v.at[1], isem)

        @pl.when(s < _NS - 1)
        def _():
            pltpu.sync_copy(zeros_hbm.at[pl.ds(0, _RPT)],
                            acc.at[pl.ds(r0, _RPT)])

        @pl.when(s == _NS - 1)
        def _():
            pltpu.sync_copy(zeros_hbm,
                            acc.at[pl.ds(15 * _RPT, _RPT_LAST)])
        plsc.subcore_barrier()

        @pl.loop(0, _NGRP)
        def _(g):
            gp = g % 2
            gq = (g + 1) % 2
            for k in range(_GROUP):
                b = (g + k) % 3
                b2 = (g + k + 2) % 3
                pltpu.make_async_copy(h_hbm.at[src_v.at[gp].at[k]],
                                      rows_v.at[b], rsem).wait()
                if k < _GROUP - 2:
                    pltpu.async_copy(h_hbm.at[src_v.at[gp].at[k + 2]],
                                     rows_v.at[b2], rsem)
                elif k == _GROUP - 2:
                    @pl.when(g + 1 < _NGRP)
                    def _():
                        pltpu.make_async_copy(src_hbm.at[wid].at[g + 1],
                                              src_v.at[gq], isem).wait()
                        pltpu.make_async_copy(dst_hbm.at[wid].at[g + 1],
                                              dst_v.at[gq], isem).wait()
                        pltpu.async_copy(h_hbm.at[src_v.at[gq].at[0]],
                                         rows_v.at[b2], rsem)
                else:
                    @pl.when(g + 1 < _NGRP)
                    def _():
                        pltpu.async_copy(h_hbm.at[src_v.at[gq].at[1]],
                                         rows_v.at[b2], rsem)
                pltpu.sync_copy(rows_v.at[b],
                                acc.at[dst_v.at[gp].at[k]], add=True)

            @pl.when(g + 2 < _NGRP)
            def _():
                pltpu.async_copy(src_hbm.at[wid].at[g + 2], src_v.at[gp], isem)
                pltpu.async_copy(dst_hbm.at[wid].at[g + 2], dst_v.at[gp], isem)

        plsc.subcore_barrier()

        @pl.when(s < _NS - 1)
        def _():
            pltpu.sync_copy(acc.at[pl.ds(r0, _RPT)],
                            out_hbm.at[c].at[pl.ds(r0, _RPT)])

        @pl.when(s == _NS - 1)
        def _():
            pltpu.sync_copy(acc.at[pl.ds(15 * _RPT, _RPT_LAST)],
                            out_hbm.at[c].at[pl.ds(15 * _RPT, _RPT_LAST)])

    return segsum_sc


_BN = 2000
_GRID = _N // _BN


def _leaky(z):
    return jnp.where(z >= 0, z, 0.01 * z)


def _pre_body(x_ref, w_ref, b_ref, o_ref):
    o_ref[...] = lax.dot_general(
        x_ref[...], w_ref[...], (((0,), (0,)), ((), ())),
        preferred_element_type=jnp.float32) + b_ref[...]


_pre = pl.pallas_call(
    _pre_body,
    out_shape=jax.ShapeDtypeStruct((_N, _H), jnp.float32),
)


def _convbn_body(h_ref, a_ref, w1_ref, b1_ref, w2_ref, b2_ref,
                 gam_ref, bet_ref, o_ref, g_vmem, st_vmem):
    i = pl.program_id(0)

    @pl.when(i < _GRID)
    def _():
        z = h_ref[...] + a_ref[0] + a_ref[1]
        t = _leaky(jnp.dot(z, w1_ref[...], preferred_element_type=jnp.float32)
                   + b1_ref[...])
        g = (jnp.dot(t, w2_ref[...], preferred_element_type=jnp.float32)
             + b2_ref[...])
        r = pl.multiple_of(i * _BN, 8)
        g_vmem[pl.ds(r, _BN), :] = g
        st = jnp.concatenate(
            [jnp.sum(g, axis=0, keepdims=True),
             jnp.sum(g * g, axis=0, keepdims=True)], axis=0)

        @pl.when(i == 0)
        def _():
            st_vmem[...] = st

        @pl.when(i != 0)
        def _():
            st_vmem[...] += st

    @pl.when(i >= _GRID)
    def _():
        m = st_vmem[0:1, :] * (1.0 / _N)
        v = st_vmem[1:2, :] * (1.0 / _N) - m * m
        a = gam_ref[...] * lax.rsqrt(v + 1e-5)
        r = pl.multiple_of((i - _GRID) * _BN, 8)
        o_ref[...] = g_vmem[pl.ds(r, _BN), :] * a + (bet_ref[...] - m * a)


_convbn = pl.pallas_call(
    _convbn_body,
    grid=(2 * _GRID,),
    in_specs=[
        pl.BlockSpec((_BN, _H), lambda i: (jnp.where(i < _GRID, i, _GRID - 1), 0)),
        pl.BlockSpec((_NC, _BN, _H),
                     lambda i: (0, jnp.where(i < _GRID, i, _GRID - 1), 0)),
        pl.BlockSpec((_H, _H), lambda i: (0, 0)),
        pl.BlockSpec((1, _H), lambda i: (0, 0)),
        pl.BlockSpec((_H, _H), lambda i: (0, 0)),
        pl.BlockSpec((1, _H), lambda i: (0, 0)),
        pl.BlockSpec((1, _H), lambda i: (0, 0)),
        pl.BlockSpec((1, _H), lambda i: (0, 0)),
    ],
    out_specs=pl.BlockSpec((_BN, _H),
                           lambda i: (jnp.where(i < _GRID, 0, i - _GRID), 0)),
    out_shape=jax.ShapeDtypeStruct((_N, _H), jnp.float32),
    scratch_shapes=[
        pltpu.VMEM((_N, _H), jnp.float32),
        pltpu.VMEM((2, _H), jnp.float32),
    ],
)


def _final_body(h_ref, a_ref, w1_ref, b1_ref, w2_ref, b2_ref,
                wp1_ref, bp1_ref, wp2_ref, bp2_ref, o_ref):
    z = h_ref[...] + a_ref[0] + a_ref[1]
    t = _leaky(jnp.dot(z, w1_ref[...], preferred_element_type=jnp.float32)
               + b1_ref[...])
    g = jnp.dot(t, w2_ref[...], preferred_element_type=jnp.float32) + b2_ref[...]
    t2 = _leaky(jnp.dot(g, wp1_ref[...], preferred_element_type=jnp.float32)
                + bp1_ref[...])
    o_ref[...] = jnp.dot(t2, wp2_ref[...],
                         preferred_element_type=jnp.float32) + bp2_ref[...]


_final = pl.pallas_call(
    _final_body,
    grid=(_GRID,),
    in_specs=[
        pl.BlockSpec((_BN, _H), lambda i: (i, 0)),
        pl.BlockSpec((_NC, _BN, _H), lambda i: (0, i, 0)),
        pl.BlockSpec((_H, _H), lambda i: (0, 0)),
        pl.BlockSpec((1, _H), lambda i: (0, 0)),
        pl.BlockSpec((_H, _H), lambda i: (0, 0)),
        pl.BlockSpec((1, _H), lambda i: (0, 0)),
        pl.BlockSpec((_H, _H), lambda i: (0, 0)),
        pl.BlockSpec((1, _H), lambda i: (0, 0)),
        pl.BlockSpec((_H, _OUT), lambda i: (0, 0)),
        pl.BlockSpec((1, _OUT), lambda i: (0, 0)),
    ],
    out_specs=pl.BlockSpec((_BN, _OUT), lambda i: (i, 0)),
    out_shape=jax.ShapeDtypeStruct((_N, _OUT), jnp.float32),
)


def kernel(x, edge_index, W_pre, b_pre, W1_0, b1_0, W2_0, b2_0,
           W1_1, b1_1, W2_1, b2_1, W1_2, b1_2, W2_2, b2_2,
           gamma_0, beta_0, gamma_1, beta_1, Wp1, bp1, Wp2, bp2):
    src = edge_index[0].reshape(_NW, _NGRP, _GROUP, _CHUNK)
    dst = edge_index[1].reshape(_NW, _NGRP, _GROUP, _CHUNK)
    zeros = jnp.zeros((_RPT_LAST, _H), jnp.float32)
    _segsum_sc = _make_segsum_sc()

    r1 = lambda v: v.reshape(1, -1)

    h = _pre(x, W_pre, r1(b_pre))
    for W1, b1, W2, b2, gam, bet in (
        (W1_0, b1_0, W2_0, b2_0, gamma_0, beta_0),
        (W1_1, b1_1, W2_1, b2_1, gamma_1, beta_1),
    ):
        agg = _segsum_sc(h, src, dst, zeros)
        h = _convbn(h, agg, W1, r1(b1), W2, r1(b2), r1(gam), r1(bet))

    agg = _segsum_sc(h, src, dst, zeros)
    out = _final(h, agg, W1_2, r1(b1_2), W2_2, r1(b2_2),
                 Wp1, r1(bp1), Wp2, r1(bp2))
    return out.reshape(1, -1)


# speedup vs baseline: 1.2799x; 1.0075x over previous
import functools

import jax
import jax.numpy as jnp
from jax import lax
from jax.experimental import pallas as pl
from jax.experimental.pallas import tpu as pltpu
from jax.experimental.pallas import tpu_sc as plsc

_N = 10000
_E = 320000
_FIN = 128
_H = 128
_OUT = 8

_NC = 2
_NS = 16
_NW = _NC * _NS
_EPW = _E // _NW
_CHUNK = 100
_NCHUNK = _EPW // _CHUNK
_GROUP = 10
_NGRP = _NCHUNK // _GROUP
_RPT = 624
_RPT_LAST = _N - 15 * _RPT

@functools.cache
def _make_segsum_sc():
    mesh = plsc.VectorSubcoreMesh(
        core_axis_name="c", subcore_axis_name="s",
        num_cores=_NC, num_subcores=_NS)

    @functools.partial(
        pl.kernel,
        out_type=jax.ShapeDtypeStruct((_NC, _N, _H), jnp.float32),
        mesh=mesh,
        scratch_types=[
            pltpu.VMEM((2, _GROUP, _CHUNK), jnp.int32),
            pltpu.VMEM((2, _GROUP, _CHUNK), jnp.int32),
            pltpu.VMEM((3, _CHUNK, _H), jnp.float32),
            pltpu.VMEM_SHARED((_N, _H), jnp.float32),
            pltpu.SemaphoreType.DMA,
            pltpu.SemaphoreType.DMA,
        ],
    )
    def segsum_sc(h_hbm, src_hbm, dst_hbm, zeros_hbm, out_hbm,
                  src_v, dst_v, rows_v, acc, rsem, isem):
        c = lax.axis_index("c")
        s = lax.axis_index("s")
        wid = s * _NC + c
        r0 = pl.multiple_of(s * _RPT, 8)

        pltpu.sync_copy(src_hbm.at[wid].at[0], src_v.at[0])
        pltpu.sync_copy(dst_hbm.at[wid].at[0], dst_v.at[0])
        pltpu.async_copy(h_hbm.at[src_v.at[0].at[0]], rows_v.at[0], rsem)
        pltpu.async_copy(h_hbm.at[src_v.at[0].at[1]], rows_v.at[1], rsem)
        pltpu.async_copy(src_hbm.at[wid].at[1], src_v.at[1], isem)
        pltpu.async_copy(dst_hbm.at[wid].at[1], dst_v.at[1], isem)

        @pl.when(s < _NS - 1)
        def _():
            pltpu.sync_copy(zeros_hbm.at[pl.ds(0, _RPT)],
                            acc.at[pl.ds(r0, _RPT)])

        @pl.when(s == _NS - 1)
        def _():
            pltpu.sync_copy(zeros_hbm,
                            acc.at[pl.ds(15 * _RPT, _RPT_LAST)])
        plsc.subcore_barrier()

        @pl.loop(0, _NGRP)
        def _(g):
            gp = g % 2
            gq = (g + 1) % 2
            for k in range(_GROUP):
                b = (g + k) % 3
                b2 = (g + k + 2) % 3
                pltpu.make_async_copy(h_hbm.at[src_v.at[gp].at[k]],
                                      rows_v.at[b], rsem).wait()
                if k < _GROUP - 2:
                    pltpu.async_copy(h_hbm.at[src_v.at[gp].at[k + 2]],
                                     rows_v.at[b2], rsem)
                elif k == _GROUP - 2:
                    @pl.when(g + 1 < _NGRP)
                    def _():
                        pltpu.make_async_copy(src_hbm.at[wid].at[g + 1],
                                              src_v.at[gq], isem).wait()
                        pltpu.make_async_copy(dst_hbm.at[wid].at[g + 1],
                                              dst_v.at[gq], isem).wait()
                        pltpu.async_copy(h_hbm.at[src_v.at[gq].at[0]],
                                         rows_v.at[b2], rsem)
                else:
                    @pl.when(g + 1 < _NGRP)
                    def _():
                        pltpu.async_copy(h_hbm.at[src_v.at[gq].at[1]],
                                         rows_v.at[b2], rsem)
                pltpu.sync_copy(rows_v.at[b],
                                acc.at[dst_v.at[gp].at[k]], add=True)

            @pl.when(g + 2 < _NGRP)
            def _():
                pltpu.async_copy(src_hbm.at[wid].at[g + 2], src_v.at[gp], isem)
                pltpu.async_copy(dst_hbm.at[wid].at[g + 2], dst_v.at[gp], isem)

        plsc.subcore_barrier()

        @pl.when(s < _NS - 1)
        def _():
            pltpu.sync_copy(acc.at[pl.ds(r0, _RPT)],
                            out_hbm.at[c].at[pl.ds(r0, _RPT)])

        @pl.when(s == _NS - 1)
        def _():
            pltpu.sync_copy(acc.at[pl.ds(15 * _RPT, _RPT_LAST)],
                            out_hbm.at[c].at[pl.ds(15 * _RPT, _RPT_LAST)])

    return segsum_sc


_BN = 5000
_GRID = _N // _BN


def _leaky(z):
    return jnp.where(z >= 0, z, 0.01 * z)


def _pre_body(x_ref, w_ref, b_ref, o_ref):
    o_ref[...] = lax.dot_general(
        x_ref[...], w_ref[...], (((0,), (0,)), ((), ())),
        preferred_element_type=jnp.float32) + b_ref[...]


_pre = pl.pallas_call(
    _pre_body,
    out_shape=jax.ShapeDtypeStruct((_N, _H), jnp.float32),
)


def _convbn_body(h_ref, a_ref, w1_ref, b1_ref, w2_ref, b2_ref,
                 gam_ref, bet_ref, o_ref, g_vmem, st_vmem):
    i = pl.program_id(0)

    @pl.when(i < _GRID)
    def _():
        z = h_ref[...] + a_ref[0] + a_ref[1]
        t = _leaky(jnp.dot(z, w1_ref[...], preferred_element_type=jnp.float32)
                   + b1_ref[...])
        g = (jnp.dot(t, w2_ref[...], preferred_element_type=jnp.float32)
             + b2_ref[...])
        r = pl.multiple_of(i * _BN, 8)
        g_vmem[pl.ds(r, _BN), :] = g
        st = jnp.concatenate(
            [jnp.sum(g, axis=0, keepdims=True),
             jnp.sum(g * g, axis=0, keepdims=True)], axis=0)

        @pl.when(i == 0)
        def _():
            st_vmem[...] = st

        @pl.when(i != 0)
        def _():
            st_vmem[...] += st

    @pl.when(i >= _GRID)
    def _():
        m = st_vmem[0:1, :] * (1.0 / _N)
        v = st_vmem[1:2, :] * (1.0 / _N) - m * m
        a = gam_ref[...] * lax.rsqrt(v + 1e-5)
        r = pl.multiple_of((i - _GRID) * _BN, 8)
        o_ref[...] = g_vmem[pl.ds(r, _BN), :] * a + (bet_ref[...] - m * a)


_convbn = pl.pallas_call(
    _convbn_body,
    grid=(2 * _GRID,),
    in_specs=[
        pl.BlockSpec((_BN, _H), lambda i: (jnp.where(i < _GRID, i, _GRID - 1), 0)),
        pl.BlockSpec((_NC, _BN, _H),
                     lambda i: (0, jnp.where(i < _GRID, i, _GRID - 1), 0)),
        pl.BlockSpec((_H, _H), lambda i: (0, 0)),
        pl.BlockSpec((1, _H), lambda i: (0, 0)),
        pl.BlockSpec((_H, _H), lambda i: (0, 0)),
        pl.BlockSpec((1, _H), lambda i: (0, 0)),
        pl.BlockSpec((1, _H), lambda i: (0, 0)),
        pl.BlockSpec((1, _H), lambda i: (0, 0)),
    ],
    out_specs=pl.BlockSpec((_BN, _H),
                           lambda i: (jnp.where(i < _GRID, 0, i - _GRID), 0)),
    out_shape=jax.ShapeDtypeStruct((_N, _H), jnp.float32),
    scratch_shapes=[
        pltpu.VMEM((_N, _H), jnp.float32),
        pltpu.VMEM((2, _H), jnp.float32),
    ],
)


def _final_body(h_ref, a_ref, w1_ref, b1_ref, w2_ref, b2_ref,
                wp1_ref, bp1_ref, wp2_ref, bp2_ref, o_ref):
    z = h_ref[...] + a_ref[0] + a_ref[1]
    t = _leaky(jnp.dot(z, w1_ref[...], preferred_element_type=jnp.float32)
               + b1_ref[...])
    g = jnp.dot(t, w2_ref[...], preferred_element_type=jnp.float32) + b2_ref[...]
    t2 = _leaky(jnp.dot(g, wp1_ref[...], preferred_element_type=jnp.float32)
                + bp1_ref[...])
    o_ref[...] = jnp.dot(t2, wp2_ref[...],
                         preferred_element_type=jnp.float32) + bp2_ref[...]


_final = pl.pallas_call(
    _final_body,
    grid=(_GRID,),
    in_specs=[
        pl.BlockSpec((_BN, _H), lambda i: (i, 0)),
        pl.BlockSpec((_NC, _BN, _H), lambda i: (0, i, 0)),
        pl.BlockSpec((_H, _H), lambda i: (0, 0)),
        pl.BlockSpec((1, _H), lambda i: (0, 0)),
        pl.BlockSpec((_H, _H), lambda i: (0, 0)),
        pl.BlockSpec((1, _H), lambda i: (0, 0)),
        pl.BlockSpec((_H, _H), lambda i: (0, 0)),
        pl.BlockSpec((1, _H), lambda i: (0, 0)),
        pl.BlockSpec((_H, _OUT), lambda i: (0, 0)),
        pl.BlockSpec((1, _OUT), lambda i: (0, 0)),
    ],
    out_specs=pl.BlockSpec((_BN, _OUT), lambda i: (i, 0)),
    out_shape=jax.ShapeDtypeStruct((_N, _OUT), jnp.float32),
)


def kernel(x, edge_index, W_pre, b_pre, W1_0, b1_0, W2_0, b2_0,
           W1_1, b1_1, W2_1, b2_1, W1_2, b1_2, W2_2, b2_2,
           gamma_0, beta_0, gamma_1, beta_1, Wp1, bp1, Wp2, bp2):
    src = edge_index[0].reshape(_NW, _NGRP, _GROUP, _CHUNK)
    dst = edge_index[1].reshape(_NW, _NGRP, _GROUP, _CHUNK)
    zeros = jnp.zeros((_RPT_LAST, _H), jnp.float32)
    _segsum_sc = _make_segsum_sc()

    r1 = lambda v: v.reshape(1, -1)

    h = _pre(x, W_pre, r1(b_pre))
    for W1, b1, W2, b2, gam, bet in (
        (W1_0, b1_0, W2_0, b2_0, gamma_0, beta_0),
        (W1_1, b1_1, W2_1, b2_1, gamma_1, beta_1),
    ):
        agg = _segsum_sc(h, src, dst, zeros)
        h = _convbn(h, agg, W1, r1(b1), W2, r1(b2), r1(gam), r1(bet))

    agg = _segsum_sc(h, src, dst, zeros)
    out = _final(h, agg, W1_2, r1(b1_2), W2_2, r1(b2_2),
                 Wp1, r1(bp1), Wp2, r1(bp2))
    return out.reshape(1, -1)
